# baseline (speedup 1.0000x reference)
import functools

import jax
import jax.numpy as jnp
from jax import lax
from jax.experimental import pallas as pl
import jax.experimental.pallas.tpu as pltpu
from jax.experimental.pallas import tpu_sc as plsc

LENGTH = 2048
CHANNEL = 1024
N_SEL = 8
NUM_HEADS = 4
NUM_FREQ = 16
BATCH = 4
CHUNK = CHANNEL // N_SEL

ROWS = 1024
QUARTER = LENGTH // 4


def _sc_selector(selw_hbm, hw_hbm, base_hbm, w_hbm,
                 selw_v, hw_v, base_v, perm_v, vals_v, hwsm_v, out_v):
    info = plsc.get_sparse_core_info()
    nc = info.num_cores
    wid = lax.axis_index("s") * nc + lax.axis_index("c")
    k = wid // 4
    q = wid % 4

    hw_v[...] = jnp.zeros((NUM_FREQ,), jnp.float32)
    pltpu.sync_copy(hw_hbm, hw_v.at[pl.ds(0, NUM_HEADS)])
    pltpu.sync_copy(selw_hbm, selw_v)
    pltpu.sync_copy(base_hbm.at[:, pl.ds(q * QUARTER, QUARTER)], base_v)

    iota16 = lax.iota(jnp.int32, 16)
    kk = jnp.full((16,), k, jnp.int32)

    hwv = hw_v[...]
    m = iota16 < NUM_HEADS
    hmax = jnp.max(jnp.where(m, hwv, jnp.float32(-1e30)), axis=0)
    he = jnp.where(m, jnp.exp(hwv - hmax), jnp.float32(0.0))
    hw_sm = he / jnp.sum(he, axis=0)

    acc = [jnp.zeros((16,), jnp.float32) for _ in range(QUARTER // 16)]
    for h in range(NUM_HEADS):
        logits = selw_v[h] * jnp.float32(h + 1)
        mx = jnp.max(logits, axis=0)
        e = jnp.exp(logits - mx)
        nw = e / jnp.sum(e, axis=0)
        vals, perm = plsc.sort_key_val(nw, iota16, descending=True)
        perm_v[...] = perm
        vals_v[...] = vals
        idxsplat = plsc.load_gather(perm_v, [kk])
        vsplat = plsc.load_gather(vals_v, [kk])
        hscal = jnp.sum(jnp.where(iota16 == h, hw_sm, jnp.float32(0.0)),
                        axis=0)
        scale = hscal * vsplat
        for c in range(QUARTER // 16):
            lane = iota16 + 16 * c
            row16 = plsc.load_gather(base_v, [idxsplat, lane])
            acc[c] = acc[c] + scale * row16
    for c in range(QUARTER // 16):
        out_v[pl.ds(16 * c, 16)] = acc[c]
    pltpu.sync_copy(out_v, w_hbm.at[k, pl.ds(q * QUARTER, QUARTER)])


def _selector_w(sel_weights, head_weights, base_weight):
    mesh = plsc.VectorSubcoreMesh(core_axis_name="c", subcore_axis_name="s")
    kfn = functools.partial(
        pl.kernel,
        mesh=mesh,
        out_type=jax.ShapeDtypeStruct((N_SEL, LENGTH), jnp.float32),
        scratch_types=[
            pltpu.VMEM((NUM_HEADS, NUM_FREQ), jnp.float32),
            pltpu.VMEM((NUM_FREQ,), jnp.float32),
            pltpu.VMEM((NUM_FREQ, QUARTER), jnp.float32),
            pltpu.VMEM((NUM_FREQ,), jnp.int32),
            pltpu.VMEM((NUM_FREQ,), jnp.float32),
            pltpu.VMEM((NUM_FREQ,), jnp.float32),
            pltpu.VMEM((QUARTER,), jnp.float32),
        ],
        compiler_params=pltpu.CompilerParams(needs_layout_passes=False),
    )(_sc_selector)
    return kfn(sel_weights, head_weights, base_weight)


def _reduce_kernel(x_ref, w_ref, out_ref):
    kblk = pl.program_id(1)
    for j in range(ROWS // CHUNK):
        wrow = w_ref[kblk * (ROWS // CHUNK) + j, :]
        xsub = x_ref[0, pl.ds(j * CHUNK, CHUNK), :]
        out_ref[0, 0, 0, pl.ds(j * CHUNK, CHUNK)] = jnp.sum(
            xsub * wrow[None, :], axis=1)


@jax.jit
def kernel(x, sel_weights, head_weights, base_weight):
    w = _selector_w(sel_weights, head_weights, base_weight)
    out = pl.pallas_call(
        _reduce_kernel,
        grid=(BATCH, CHANNEL // ROWS),
        in_specs=[
            pl.BlockSpec((1, ROWS, LENGTH), lambda b, k: (b, k, 0)),
            pl.BlockSpec((N_SEL, LENGTH), lambda b, k: (0, 0)),
        ],
        out_specs=pl.BlockSpec((1, 1, 1, ROWS), lambda b, k: (b, k, 0, 0)),
        out_shape=jax.ShapeDtypeStruct((BATCH, CHANNEL // ROWS, 1, ROWS),
                                       jnp.float32),
    )(x, w)
    return out.reshape(BATCH, CHANNEL)


# speedup vs baseline: 1.0775x; 1.0775x over previous
import functools

import jax
import jax.numpy as jnp
from jax import lax
from jax.experimental import pallas as pl
import jax.experimental.pallas.tpu as pltpu
from jax.experimental.pallas import tpu_sc as plsc

LENGTH = 2048
CHANNEL = 1024
N_SEL = 8
NUM_HEADS = 4
NUM_FREQ = 16
BATCH = 4
CHUNK = CHANNEL // N_SEL

ROWS = 1024


def _sc_selector(selw_hbm, hw_hbm, coeff_hbm,
                 selw_v, hw_v, perm_v, vals_v, out_v):
    info = plsc.get_sparse_core_info()
    nc = info.num_cores
    wid = lax.axis_index("s") * nc + lax.axis_index("c")

    @pl.when(wid < N_SEL)
    def _():
        k = wid

        hw_v[...] = jnp.zeros((NUM_FREQ,), jnp.float32)
        pltpu.sync_copy(hw_hbm, hw_v.at[pl.ds(0, NUM_HEADS)])
        pltpu.sync_copy(selw_hbm, selw_v)

        iota16 = lax.iota(jnp.int32, 16)
        kk = jnp.full((16,), k, jnp.int32)

        hwv = hw_v[...]
        m = iota16 < NUM_HEADS
        hmax = jnp.max(jnp.where(m, hwv, jnp.float32(-1e30)), axis=0)
        he = jnp.where(m, jnp.exp(hwv - hmax), jnp.float32(0.0))
        hw_sm = he / jnp.sum(he, axis=0)

        acc = jnp.zeros((16,), jnp.float32)
        for h in range(NUM_HEADS):
            logits = selw_v[h] * jnp.float32(h + 1)
            mx = jnp.max(logits, axis=0)
            e = jnp.exp(logits - mx)
            nw = e / jnp.sum(e, axis=0)
            vals, perm = plsc.sort_key_val(nw, iota16, descending=True)
            perm_v[...] = perm
            vals_v[...] = vals
            idxsplat = plsc.load_gather(perm_v, [kk])
            vsplat = plsc.load_gather(vals_v, [kk])
            hscal = jnp.sum(jnp.where(iota16 == h, hw_sm, jnp.float32(0.0)),
                            axis=0)
            onehot = jnp.where(iota16 == idxsplat, jnp.float32(1.0),
                               jnp.float32(0.0))
            acc = acc + hscal * vsplat * onehot
        out_v[...] = acc
        pltpu.sync_copy(out_v, coeff_hbm.at[k])


def _selector_coeff(sel_weights, head_weights):
    mesh = plsc.VectorSubcoreMesh(core_axis_name="c", subcore_axis_name="s")
    kfn = functools.partial(
        pl.kernel,
        mesh=mesh,
        out_type=jax.ShapeDtypeStruct((N_SEL, NUM_FREQ), jnp.float32),
        scratch_types=[
            pltpu.VMEM((NUM_HEADS, NUM_FREQ), jnp.float32),
            pltpu.VMEM((NUM_FREQ,), jnp.float32),
            pltpu.VMEM((NUM_FREQ,), jnp.int32),
            pltpu.VMEM((NUM_FREQ,), jnp.float32),
            pltpu.VMEM((NUM_FREQ,), jnp.float32),
        ],
        compiler_params=pltpu.CompilerParams(needs_layout_passes=False),
    )(_sc_selector)
    return kfn(sel_weights, head_weights)


def _proj_kernel(x_ref, base_ref, p_ref):
    xblk = x_ref[0]
    p_ref[0] = lax.dot_general(
        xblk, base_ref[...],
        dimension_numbers=(((1,), (1,)), ((), ())),
        preferred_element_type=jnp.float32)


def _combine_kernel(p_ref, coeff_ref, out_ref):
    coeff = coeff_ref[...]
    ce = jnp.broadcast_to(coeff[:, None, :], (N_SEL, CHUNK, NUM_FREQ))
    ce = ce.reshape(CHANNEL, NUM_FREQ)
    for b in range(BATCH):
        out_ref[b, 0, 0, :] = jnp.sum(p_ref[b] * ce, axis=1)


@jax.jit
def kernel(x, sel_weights, head_weights, base_weight):
    coeff = _selector_coeff(sel_weights, head_weights)

    p = pl.pallas_call(
        _proj_kernel,
        grid=(BATCH, CHANNEL // ROWS),
        in_specs=[
            pl.BlockSpec((1, ROWS, LENGTH), lambda b, k: (b, k, 0)),
            pl.BlockSpec((NUM_FREQ, LENGTH), lambda b, k: (0, 0)),
        ],
        out_specs=pl.BlockSpec((1, ROWS, NUM_FREQ), lambda b, k: (b, k, 0)),
        out_shape=jax.ShapeDtypeStruct((BATCH, CHANNEL, NUM_FREQ),
                                       jnp.float32),
    )(x, base_weight)

    out = pl.pallas_call(
        _combine_kernel,
        in_specs=[
            pl.BlockSpec((BATCH, CHANNEL, NUM_FREQ), lambda: (0, 0, 0)),
            pl.BlockSpec((N_SEL, NUM_FREQ), lambda: (0, 0)),
        ],
        out_specs=pl.BlockSpec((BATCH, 1, 1, CHANNEL), lambda: (0, 0, 0, 0)),
        out_shape=jax.ShapeDtypeStruct((BATCH, 1, 1, CHANNEL), jnp.float32),
    )(p, coeff)
    return out.reshape(BATCH, CHANNEL)
